# baseline (speedup 1.0000x reference)
import jax
import jax.numpy as jnp
from jax import lax
from jax.experimental import pallas as pl
from jax.experimental.pallas import tpu as pltpu
from jax.experimental.pallas import tpu_sc as plsc

N = 10000
E = 320000
DIN = 128
DE = 16
DOUT = 128

NC = 2
NS = 16
NW = NC * NS

CHUNK = 128
E_PER_W = 10240
EPAD = E_PER_W * NW
N_FULL = E_PER_W // CHUNK
NPAD = 10240
ROWS_PER_TILE = NPAD // NS

NB = 2
NI = 4


def _scn_body(nf_hbm, src_hbm, dst_hbm, z128_hbm,
              accn_out,
              sidx_v, didx_v, nrows_v, accn_s, sem):
    cid = lax.axis_index("c")
    sid = lax.axis_index("s")
    wid = sid * NC + cid
    base = wid * E_PER_W
    row0 = sid * ROWS_PER_TILE

    pltpu.sync_copy(z128_hbm, accn_s.at[pl.ds(row0, ROWS_PER_TILE), :])
    plsc.subcore_barrier()

    def _chunk(c, _):
        off = base + c * CHUNK
        pltpu.sync_copy(src_hbm.at[pl.ds(off, CHUNK)], sidx_v)
        pltpu.sync_copy(dst_hbm.at[pl.ds(off, CHUNK)], didx_v)
        pltpu.async_copy(nf_hbm.at[sidx_v], nrows_v, sem).wait()
        pltpu.sync_copy(nrows_v, accn_s.at[didx_v], add=True)
        return 0
    lax.fori_loop(0, N_FULL, _chunk, 0)

    plsc.subcore_barrier()
    pltpu.sync_copy(accn_s.at[pl.ds(row0, ROWS_PER_TILE), :],
                    accn_out.at[cid, pl.ds(row0, ROWS_PER_TILE), :])


def _sce_body(ef8_hbm, eidx_hbm, z128_hbm, tmpl_hbm, dep_hbm,
              accec_out,
              didx_v, erows_v, comb_v, accec_s, semi, sems):
    cid = lax.axis_index("c")
    sid = lax.axis_index("s")
    wid = sid * NC + cid
    base = wid * E_PER_W
    row0 = sid * ROWS_PER_TILE

    pltpu.sync_copy(z128_hbm, accec_s.at[pl.ds(row0, ROWS_PER_TILE), :])
    for b4 in range(NB):
        pltpu.sync_copy(tmpl_hbm, comb_v.at[b4])
    plsc.subcore_barrier()

    def in_start(c, s8):
        off = base + c * CHUNK
        row_off = wid * (E_PER_W // 8) + c * (CHUNK // 8)
        pltpu.async_copy(eidx_hbm.at[1, pl.ds(off, CHUNK)],
                         didx_v.at[s8], semi.at[s8])
        pltpu.async_copy(ef8_hbm.at[pl.ds(row_off, CHUNK // 8), :],
                         erows_v.at[s8], semi.at[s8])

    def in_wait(s8):
        pltpu.make_async_copy(eidx_hbm.at[1, pl.ds(0, CHUNK)],
                              didx_v.at[s8], semi.at[s8]).wait()
        pltpu.make_async_copy(ef8_hbm.at[pl.ds(0, CHUNK // 8), :],
                              erows_v.at[s8], semi.at[s8]).wait()

    def sca_start(s8, b4):
        pltpu.async_copy(comb_v.at[b4], accec_s.at[didx_v.at[s8]],
                         sems.at[b4], add=True)

    def sca_wait(s8, b4):
        pltpu.make_async_copy(comb_v.at[b4], accec_s.at[didx_v.at[s8]],
                              sems.at[b4]).wait()

    def body(c, b, first, last):
        b2, b4 = b % NB, b % NI
        in_wait(b4)
        if not (first and b < 2):
            sca_wait((b4 + 2) % NI, b2)
        if not (last and b >= 2):
            in_start(c + 2, (b4 + 2) % NI)

        def _cp(r, _):
            for k in range(8):
                comb_v[b2, r * 8 + k, :DE] = \
                    erows_v[b4, r, k * DE:(k + 1) * DE]
            return 0
        lax.fori_loop(0, CHUNK // 8, _cp, 0, unroll=2)
        sca_start(b4, b2)

    for b in range(NB):
        in_start(b, b)
    for b in range(NI):
        body(b, b, True, False)

    def outer(g, _):
        for b in range(NI):
            body(g * NI + b, b, False, False)
        return 0
    lax.fori_loop(1, N_FULL // NI - 1, outer, 0)

    for b in range(NI):
        body(N_FULL - NI + b, b, False, True)

    for b2 in range(NB):
        sca_wait((N_FULL - NB + b2) % NI, (N_FULL - NB + b2) % NB)

    plsc.subcore_barrier()
    pltpu.sync_copy(accec_s.at[pl.ds(row0, ROWS_PER_TILE), :],
                    accec_out.at[cid, pl.ds(row0, ROWS_PER_TILE), :])


def _sc_segment_sums(nf, ef, srcp, dstp, eidx, z128, tmpl):
    mesh = plsc.VectorSubcoreMesh(core_axis_name="c", subcore_axis_name="s")
    fn = pl.kernel(
        _scn_body,
        out_type=jax.ShapeDtypeStruct((NC, NPAD, DIN), jnp.float32),
        mesh=mesh,
        scratch_types=[
            pltpu.VMEM((CHUNK,), jnp.int32),
            pltpu.VMEM((CHUNK,), jnp.int32),
            pltpu.VMEM((CHUNK, DIN), jnp.float32),
            pltpu.VMEM_SHARED((NPAD, DIN), jnp.float32),
            pltpu.SemaphoreType.DMA,
        ],
    )
    accn = fn(nf, srcp, dstp, z128)
    fe = pl.kernel(
        _sce_body,
        out_type=jax.ShapeDtypeStruct((NC, NPAD, DIN), jnp.float32),
        mesh=mesh,
        scratch_types=[
            pltpu.VMEM((NI, CHUNK), jnp.int32),
            pltpu.VMEM((NI, CHUNK // 8, DIN), jnp.float32),
            pltpu.VMEM((NB, CHUNK, DIN), jnp.float32),
            pltpu.VMEM_SHARED((NPAD, DIN), jnp.float32),
            pltpu.SemaphoreType.DMA((NI,)),
            pltpu.SemaphoreType.DMA((NB,)),
        ],
    )
    accec = fe(ef, eidx, z128, tmpl, accn)
    return accn, accec


TC_R = 1000


def _tc_body(nf_ref, accn_ref, accec_ref,
             w1t_ref, w2t_ref, bmsg_ref, wa1t_ref, wa2t_ref, bapp_ref,
             out_ref):
    sn = accn_ref[0] + accn_ref[1]
    ec = accec_ref[0] + accec_ref[1]
    se = ec[:, :DE]
    cnt = ec[:, DE:DE + 1]
    inv = 1.0 / jnp.maximum(cnt, 1.0)
    hn = lax.dot_general(sn, w1t_ref[...], (((1,), (0,)), ((), ())),
                         precision=lax.Precision.HIGHEST)
    hn += lax.dot_general(se, w2t_ref[...], (((1,), (0,)), ((), ())),
                          precision=lax.Precision.HIGHEST)
    hn = (hn + cnt * bmsg_ref[...]) * inv
    out = lax.dot_general(nf_ref[...], wa1t_ref[...], (((1,), (0,)), ((), ())),
                          precision=lax.Precision.HIGHEST)
    out += lax.dot_general(hn, wa2t_ref[...], (((1,), (0,)), ((), ())),
                           precision=lax.Precision.HIGHEST)
    out_ref[...] = jnp.maximum(out + bapp_ref[...], 0.0)


def _tc_apply(nf, accn, accec, w1t, w2t, bmsg, wa1t, wa2t, bapp):
    zero_map = lambda i: (0, 0)
    return pl.pallas_call(
        _tc_body,
        grid=(N // TC_R,),
        in_specs=[
            pl.BlockSpec((TC_R, DIN), lambda i: (i, 0)),
            pl.BlockSpec((NC, TC_R, DIN), lambda i: (0, i, 0)),
            pl.BlockSpec((NC, TC_R, DIN), lambda i: (0, i, 0)),
            pl.BlockSpec((DIN, DOUT), zero_map),
            pl.BlockSpec((DE, DOUT), zero_map),
            pl.BlockSpec((1, DOUT), zero_map),
            pl.BlockSpec((DIN, DOUT), zero_map),
            pl.BlockSpec((DOUT, DOUT), zero_map),
            pl.BlockSpec((1, DOUT), zero_map),
        ],
        out_specs=pl.BlockSpec((TC_R, DOUT), lambda i: (i, 0)),
        out_shape=jax.ShapeDtypeStruct((N, DOUT), jnp.float32),
    )(nf, accn, accec, w1t, w2t, bmsg, wa1t, wa2t, bapp)


@jax.jit
def kernel(nfeats, efeats, edge_index, W_msg_w, W_msg_b, W_apply_w, W_apply_b):
    nf = nfeats[:, 0, :]
    ef = efeats[:, 0, :]
    npad_e = EPAD - E
    srcp = jnp.concatenate([edge_index[0], jnp.zeros((npad_e,), jnp.int32)])
    pad_dst = N + (jnp.arange(npad_e, dtype=jnp.int32) % (NPAD - N))
    dstp = jnp.concatenate([edge_index[1], pad_dst])
    eidx = jnp.stack([srcp, dstp])
    efp = jnp.concatenate([ef, jnp.zeros((npad_e, DE), jnp.float32)])
    ef8 = efp.reshape(EPAD // 8, 8 * DE)
    z128 = jnp.zeros((ROWS_PER_TILE, DIN), jnp.float32)
    tmpl = jnp.zeros((CHUNK, DIN), jnp.float32).at[:, DE].set(1.0)
    accn, accec = _sc_segment_sums(nf, ef8, srcp, dstp, eidx, z128, tmpl)
    w1t = W_msg_w[:, :DIN].T
    w2t = W_msg_w[:, DIN:].T
    wa1t = W_apply_w[:, :DIN].T
    wa2t = W_apply_w[:, DIN:].T
    out = _tc_apply(nf, accn, accec, w1t, w2t,
                    W_msg_b[None, :], wa1t, wa2t, W_apply_b[None, :])
    return out[:, None, :]


# speedup vs baseline: 1.5388x; 1.5388x over previous
import jax
import jax.numpy as jnp
from jax import lax
from jax.experimental import pallas as pl
from jax.experimental.pallas import tpu as pltpu
from jax.experimental.pallas import tpu_sc as plsc

N = 10000
E = 320000
DIN = 128
DE = 16
DOUT = 128

NC = 2
NS = 16
NW = NC * NS

E_PER_W = E // NW
CHUNK = 128
N_FULL = E_PER_W // CHUNK
REM = E_PER_W - N_FULL * CHUNK
NPAD = 10240
ROWS_PER_TILE = NPAD // NS


def _scn_body(nf_hbm, src_hbm, dst_hbm, z128_hbm,
              accn_out,
              sidx_v, didx_v, nrows_v, sidx_r, didx_r, nrows_r,
              accn_s, sem):
    cid = lax.axis_index("c")
    sid = lax.axis_index("s")
    wid = sid * NC + cid
    base = wid * E_PER_W
    row0 = sid * ROWS_PER_TILE

    pltpu.sync_copy(z128_hbm, accn_s.at[pl.ds(row0, ROWS_PER_TILE), :])
    plsc.subcore_barrier()

    def _chunk(c, _):
        off = base + c * CHUNK
        pltpu.sync_copy(src_hbm.at[pl.ds(off, CHUNK)], sidx_v)
        pltpu.sync_copy(dst_hbm.at[pl.ds(off, CHUNK)], didx_v)
        pltpu.async_copy(nf_hbm.at[sidx_v], nrows_v, sem).wait()
        pltpu.sync_copy(nrows_v, accn_s.at[didx_v], add=True)
        return 0
    lax.fori_loop(0, N_FULL, _chunk, 0)

    off = base + N_FULL * CHUNK
    pltpu.sync_copy(src_hbm.at[pl.ds(off, REM)], sidx_r)
    pltpu.sync_copy(dst_hbm.at[pl.ds(off, REM)], didx_r)
    pltpu.async_copy(nf_hbm.at[sidx_r], nrows_r, sem).wait()
    pltpu.sync_copy(nrows_r, accn_s.at[didx_r], add=True)

    plsc.subcore_barrier()
    pltpu.sync_copy(accn_s.at[pl.ds(row0, ROWS_PER_TILE), :],
                    accn_out.at[cid, pl.ds(row0, ROWS_PER_TILE), :])


def _sce_body(ef_hbm, dst_hbm, z128_hbm, tmpl_hbm,
              accec_out,
              didx_v, comb_v, erows_v, didx_r, comb_r, erows_r,
              accec_s, sem):
    cid = lax.axis_index("c")
    sid = lax.axis_index("s")
    wid = sid * NC + cid
    base = wid * E_PER_W
    row0 = sid * ROWS_PER_TILE

    pltpu.sync_copy(z128_hbm, accec_s.at[pl.ds(row0, ROWS_PER_TILE), :])
    pltpu.sync_copy(tmpl_hbm, comb_v)
    pltpu.sync_copy(tmpl_hbm.at[pl.ds(0, REM), :], comb_r)
    plsc.subcore_barrier()

    def _chunk(c, _):
        off = base + c * CHUNK
        pltpu.sync_copy(dst_hbm.at[pl.ds(off, CHUNK)], didx_v)
        pltpu.sync_copy(ef_hbm.at[pl.ds(off, CHUNK), :], erows_v)

        def _cp(i, _):
            comb_v[i, :DE] = erows_v[i, :]
            return 0
        lax.fori_loop(0, CHUNK, _cp, 0, unroll=8)
        pltpu.sync_copy(comb_v, accec_s.at[didx_v], add=True)
        return 0
    lax.fori_loop(0, N_FULL, _chunk, 0)

    off = base + N_FULL * CHUNK
    pltpu.sync_copy(dst_hbm.at[pl.ds(off, REM)], didx_r)
    pltpu.sync_copy(ef_hbm.at[pl.ds(off, REM), :], erows_r)

    def _cpr(i, _):
        comb_r[i, :DE] = erows_r[i, :]
        return 0
    lax.fori_loop(0, REM, _cpr, 0, unroll=8)
    pltpu.sync_copy(comb_r, accec_s.at[didx_r], add=True)

    plsc.subcore_barrier()
    pltpu.sync_copy(accec_s.at[pl.ds(row0, ROWS_PER_TILE), :],
                    accec_out.at[cid, pl.ds(row0, ROWS_PER_TILE), :])


def _sc_segment_sums(nf, ef, src, dst, z128, tmpl):
    mesh = plsc.VectorSubcoreMesh(core_axis_name="c", subcore_axis_name="s")
    fn = pl.kernel(
        _scn_body,
        out_type=jax.ShapeDtypeStruct((NC, NPAD, DIN), jnp.float32),
        mesh=mesh,
        scratch_types=[
            pltpu.VMEM((CHUNK,), jnp.int32),
            pltpu.VMEM((CHUNK,), jnp.int32),
            pltpu.VMEM((CHUNK, DIN), jnp.float32),
            pltpu.VMEM((REM,), jnp.int32),
            pltpu.VMEM((REM,), jnp.int32),
            pltpu.VMEM((REM, DIN), jnp.float32),
            pltpu.VMEM_SHARED((NPAD, DIN), jnp.float32),
            pltpu.SemaphoreType.DMA,
        ],
    )
    accn = fn(nf, src, dst, z128)
    fe = pl.kernel(
        _sce_body,
        out_type=jax.ShapeDtypeStruct((NC, NPAD, DIN), jnp.float32),
        mesh=mesh,
        scratch_types=[
            pltpu.VMEM((CHUNK,), jnp.int32),
            pltpu.VMEM((CHUNK, DIN), jnp.float32),
            pltpu.VMEM((CHUNK, DE), jnp.float32),
            pltpu.VMEM((REM,), jnp.int32),
            pltpu.VMEM((REM, DIN), jnp.float32),
            pltpu.VMEM((REM, DE), jnp.float32),
            pltpu.VMEM_SHARED((NPAD, DIN), jnp.float32),
            pltpu.SemaphoreType.DMA,
        ],
    )
    accec = fe(ef, dst, z128, tmpl)
    return accn, accec


TC_R = 1000


def _tc_body(nf_ref, accn_ref, accec_ref,
             w1t_ref, w2t_ref, bmsg_ref, wa1t_ref, wa2t_ref, bapp_ref,
             out_ref):
    sn = accn_ref[0] + accn_ref[1]
    ec = accec_ref[0] + accec_ref[1]
    se = ec[:, :DE]
    cnt = ec[:, DE:DE + 1]
    inv = 1.0 / jnp.maximum(cnt, 1.0)
    hn = lax.dot_general(sn, w1t_ref[...], (((1,), (0,)), ((), ())),
                         precision=lax.Precision.HIGHEST)
    hn += lax.dot_general(se, w2t_ref[...], (((1,), (0,)), ((), ())),
                          precision=lax.Precision.HIGHEST)
    hn = (hn + cnt * bmsg_ref[...]) * inv
    out = lax.dot_general(nf_ref[...], wa1t_ref[...], (((1,), (0,)), ((), ())),
                          precision=lax.Precision.HIGHEST)
    out += lax.dot_general(hn, wa2t_ref[...], (((1,), (0,)), ((), ())),
                           precision=lax.Precision.HIGHEST)
    out_ref[...] = jnp.maximum(out + bapp_ref[...], 0.0)


def _tc_apply(nf, accn, accec, w1t, w2t, bmsg, wa1t, wa2t, bapp):
    zero_map = lambda i: (0, 0)
    return pl.pallas_call(
        _tc_body,
        grid=(N // TC_R,),
        in_specs=[
            pl.BlockSpec((TC_R, DIN), lambda i: (i, 0)),
            pl.BlockSpec((NC, TC_R, DIN), lambda i: (0, i, 0)),
            pl.BlockSpec((NC, TC_R, DIN), lambda i: (0, i, 0)),
            pl.BlockSpec((DIN, DOUT), zero_map),
            pl.BlockSpec((DE, DOUT), zero_map),
            pl.BlockSpec((1, DOUT), zero_map),
            pl.BlockSpec((DIN, DOUT), zero_map),
            pl.BlockSpec((DOUT, DOUT), zero_map),
            pl.BlockSpec((1, DOUT), zero_map),
        ],
        out_specs=pl.BlockSpec((TC_R, DOUT), lambda i: (i, 0)),
        out_shape=jax.ShapeDtypeStruct((N, DOUT), jnp.float32),
    )(nf, accn, accec, w1t, w2t, bmsg, wa1t, wa2t, bapp)


@jax.jit
def kernel(nfeats, efeats, edge_index, W_msg_w, W_msg_b, W_apply_w, W_apply_b):
    nf = nfeats[:, 0, :]
    ef = efeats[:, 0, :]
    src = edge_index[0]
    dst = edge_index[1]
    z128 = jnp.zeros((ROWS_PER_TILE, DIN), jnp.float32)
    tmpl = jnp.zeros((CHUNK, DIN), jnp.float32).at[:, DE].set(1.0)
    accn, accec = _sc_segment_sums(nf, ef, src, dst, z128, tmpl)
    w1t = W_msg_w[:, :DIN].T
    w2t = W_msg_w[:, DIN:].T
    wa1t = W_apply_w[:, :DIN].T
    wa2t = W_apply_w[:, DIN:].T
    out = _tc_apply(nf, accn, accec, w1t, w2t,
                    W_msg_b[None, :], wa1t, wa2t, W_apply_b[None, :])
    return out[:, None, :]


# speedup vs baseline: 1.7645x; 1.1467x over previous
import jax
import jax.numpy as jnp
from jax import lax
from jax.experimental import pallas as pl
from jax.experimental.pallas import tpu as pltpu
from jax.experimental.pallas import tpu_sc as plsc

N = 10000
E = 320000
DIN = 128
DE = 16
DOUT = 128

NC = 2
NS = 16
NW = NC * NS

CHUNK = 128
E_PER_W = 10240
EPAD = E_PER_W * NW
N_FULL = E_PER_W // CHUNK
NPAD = 10240
ROWS_PER_TILE = NPAD // NS

NB = 2
NI = 4


def _scn_body(nf_hbm, src_hbm, dst_hbm, z128_hbm,
              accn_out,
              sidx_v, didx_v, nrows_v, accn_s, sem):
    cid = lax.axis_index("c")
    sid = lax.axis_index("s")
    wid = sid * NC + cid
    base = wid * E_PER_W
    row0 = sid * ROWS_PER_TILE

    pltpu.sync_copy(z128_hbm, accn_s.at[pl.ds(row0, ROWS_PER_TILE), :])
    plsc.subcore_barrier()

    def _chunk(c, _):
        off = base + c * CHUNK
        pltpu.sync_copy(src_hbm.at[pl.ds(off, CHUNK)], sidx_v)
        pltpu.sync_copy(dst_hbm.at[pl.ds(off, CHUNK)], didx_v)
        pltpu.async_copy(nf_hbm.at[sidx_v], nrows_v, sem).wait()
        pltpu.sync_copy(nrows_v, accn_s.at[didx_v], add=True)
        return 0
    lax.fori_loop(0, N_FULL, _chunk, 0)

    plsc.subcore_barrier()
    pltpu.sync_copy(accn_s.at[pl.ds(row0, ROWS_PER_TILE), :],
                    accn_out.at[cid, pl.ds(row0, ROWS_PER_TILE), :])


def _sce_body(ef8_hbm, eidx_hbm, z128_hbm, tmpl_hbm, dep_hbm,
              accec_out,
              didx_v, erows_v, comb_v, accec_s, semi, sems):
    cid = lax.axis_index("c")
    sid = lax.axis_index("s")
    wid = sid * NC + cid
    base = wid * E_PER_W
    row0 = sid * ROWS_PER_TILE

    pltpu.sync_copy(z128_hbm, accec_s.at[pl.ds(row0, ROWS_PER_TILE), :])
    for b4 in range(NB):
        pltpu.sync_copy(tmpl_hbm, comb_v.at[b4])
    plsc.subcore_barrier()

    def in_start(c, s8):
        off = base + c * CHUNK
        row_off = wid * (E_PER_W // 8) + c * (CHUNK // 8)
        pltpu.async_copy(eidx_hbm.at[1, pl.ds(off, CHUNK)],
                         didx_v.at[s8], semi.at[s8])
        pltpu.async_copy(ef8_hbm.at[pl.ds(row_off, CHUNK // 8), :],
                         erows_v.at[s8], semi.at[s8])

    def in_wait(s8):
        pltpu.make_async_copy(eidx_hbm.at[1, pl.ds(0, CHUNK)],
                              didx_v.at[s8], semi.at[s8]).wait()
        pltpu.make_async_copy(ef8_hbm.at[pl.ds(0, CHUNK // 8), :],
                              erows_v.at[s8], semi.at[s8]).wait()

    def sca_start(s8, b4):
        pltpu.async_copy(comb_v.at[b4], accec_s.at[didx_v.at[s8]],
                         sems.at[b4], add=True)

    def sca_wait(s8, b4):
        pltpu.make_async_copy(comb_v.at[b4], accec_s.at[didx_v.at[s8]],
                              sems.at[b4]).wait()

    def body(c, b, first, last):
        b2, b4 = b % NB, b % NI
        in_wait(b4)
        if not (first and b < 2):
            sca_wait((b4 + 2) % NI, b2)
        if not (last and b >= 2):
            in_start(c + 2, (b4 + 2) % NI)

        def _cp(r, _):
            for k in range(8):
                comb_v[b2, r * 8 + k, :DE] = \
                    erows_v[b4, r, k * DE:(k + 1) * DE]
            return 0
        lax.fori_loop(0, CHUNK // 8, _cp, 0, unroll=2)
        sca_start(b4, b2)

    for b in range(NB):
        in_start(b, b)
    for b in range(NI):
        body(b, b, True, False)

    def outer(g, _):
        for b in range(NI):
            body(g * NI + b, b, False, False)
        return 0
    lax.fori_loop(1, N_FULL // NI - 1, outer, 0)

    for b in range(NI):
        body(N_FULL - NI + b, b, False, True)

    for b2 in range(NB):
        sca_wait((N_FULL - NB + b2) % NI, (N_FULL - NB + b2) % NB)

    plsc.subcore_barrier()
    pltpu.sync_copy(accec_s.at[pl.ds(row0, ROWS_PER_TILE), :],
                    accec_out.at[cid, pl.ds(row0, ROWS_PER_TILE), :])


def _sc_segment_sums(nf, ef, srcp, dstp, eidx, z128, tmpl):
    mesh = plsc.VectorSubcoreMesh(core_axis_name="c", subcore_axis_name="s")
    fn = pl.kernel(
        _scn_body,
        out_type=jax.ShapeDtypeStruct((NC, NPAD, DIN), jnp.float32),
        mesh=mesh,
        scratch_types=[
            pltpu.VMEM((CHUNK,), jnp.int32),
            pltpu.VMEM((CHUNK,), jnp.int32),
            pltpu.VMEM((CHUNK, DIN), jnp.float32),
            pltpu.VMEM_SHARED((NPAD, DIN), jnp.float32),
            pltpu.SemaphoreType.DMA,
        ],
    )
    accn = fn(nf, srcp, dstp, z128)
    fe = pl.kernel(
        _sce_body,
        out_type=jax.ShapeDtypeStruct((NC, NPAD, DIN), jnp.float32),
        mesh=mesh,
        scratch_types=[
            pltpu.VMEM((NI, CHUNK), jnp.int32),
            pltpu.VMEM((NI, CHUNK // 8, DIN), jnp.float32),
            pltpu.VMEM((NB, CHUNK, DIN), jnp.float32),
            pltpu.VMEM_SHARED((NPAD, DIN), jnp.float32),
            pltpu.SemaphoreType.DMA((NI,)),
            pltpu.SemaphoreType.DMA((NB,)),
        ],
    )
    accec = fe(ef, eidx, z128, tmpl, accn)
    return accn, accec


TC_R = 1000


def _tc_body(nf_ref, accn_ref, accec_ref,
             w1t_ref, w2t_ref, bmsg_ref, wa1t_ref, wa2t_ref, bapp_ref,
             out_ref):
    sn = accn_ref[0] + accn_ref[1]
    ec = accec_ref[0] + accec_ref[1]
    se = ec[:, :DE]
    cnt = ec[:, DE:DE + 1]
    inv = 1.0 / jnp.maximum(cnt, 1.0)
    hn = lax.dot_general(sn, w1t_ref[...], (((1,), (0,)), ((), ())),
                         precision=lax.Precision.HIGHEST)
    hn += lax.dot_general(se, w2t_ref[...], (((1,), (0,)), ((), ())),
                          precision=lax.Precision.HIGHEST)
    hn = (hn + cnt * bmsg_ref[...]) * inv
    out = lax.dot_general(nf_ref[...], wa1t_ref[...], (((1,), (0,)), ((), ())),
                          precision=lax.Precision.HIGHEST)
    out += lax.dot_general(hn, wa2t_ref[...], (((1,), (0,)), ((), ())),
                           precision=lax.Precision.HIGHEST)
    out_ref[...] = jnp.maximum(out + bapp_ref[...], 0.0)


def _tc_apply(nf, accn, accec, w1t, w2t, bmsg, wa1t, wa2t, bapp):
    zero_map = lambda i: (0, 0)
    return pl.pallas_call(
        _tc_body,
        grid=(N // TC_R,),
        in_specs=[
            pl.BlockSpec((TC_R, DIN), lambda i: (i, 0)),
            pl.BlockSpec((NC, TC_R, DIN), lambda i: (0, i, 0)),
            pl.BlockSpec((NC, TC_R, DIN), lambda i: (0, i, 0)),
            pl.BlockSpec((DIN, DOUT), zero_map),
            pl.BlockSpec((DE, DOUT), zero_map),
            pl.BlockSpec((1, DOUT), zero_map),
            pl.BlockSpec((DIN, DOUT), zero_map),
            pl.BlockSpec((DOUT, DOUT), zero_map),
            pl.BlockSpec((1, DOUT), zero_map),
        ],
        out_specs=pl.BlockSpec((TC_R, DOUT), lambda i: (i, 0)),
        out_shape=jax.ShapeDtypeStruct((N, DOUT), jnp.float32),
    )(nf, accn, accec, w1t, w2t, bmsg, wa1t, wa2t, bapp)


@jax.jit
def kernel(nfeats, efeats, edge_index, W_msg_w, W_msg_b, W_apply_w, W_apply_b):
    nf = nfeats[:, 0, :]
    ef = efeats[:, 0, :]
    npad_e = EPAD - E
    pad_src = jnp.arange(npad_e, dtype=jnp.int32) % N
    srcp = jnp.concatenate([edge_index[0], pad_src])
    pad_dst = N + (jnp.arange(npad_e, dtype=jnp.int32) % (NPAD - N))
    dstp = jnp.concatenate([edge_index[1], pad_dst])
    eidx = jnp.stack([srcp, dstp])
    efp = jnp.concatenate([ef, jnp.zeros((npad_e, DE), jnp.float32)])
    ef8 = efp.reshape(EPAD // 8, 8 * DE)
    z128 = jnp.zeros((ROWS_PER_TILE, DIN), jnp.float32)
    tmpl = jnp.zeros((CHUNK, DIN), jnp.float32).at[:, DE].set(1.0)
    accn, accec = _sc_segment_sums(nf, ef8, srcp, dstp, eidx, z128, tmpl)
    w1t = W_msg_w[:, :DIN].T
    w2t = W_msg_w[:, DIN:].T
    wa1t = W_apply_w[:, :DIN].T
    wa2t = W_apply_w[:, DIN:].T
    out = _tc_apply(nf, accn, accec, w1t, w2t,
                    W_msg_b[None, :], wa1t, wa2t, W_apply_b[None, :])
    return out[:, None, :]


# speedup vs baseline: 2.0879x; 1.1833x over previous
import jax
import jax.numpy as jnp
from jax import lax
from jax.experimental import pallas as pl
from jax.experimental.pallas import tpu as pltpu
from jax.experimental.pallas import tpu_sc as plsc

N = 10000
E = 320000
DIN = 128
DE = 16
DOUT = 128

NC = 2
NS = 16
NW = NC * NS

CHUNK = 128
E_PER_W = 10240
EPAD = E_PER_W * NW
N_FULL = E_PER_W // CHUNK
NPAD = 10240
ROWS_PER_TILE = NPAD // NS

NB = 2
NI = 4


def _scn_body(nf_hbm, src_hbm, dst_hbm, z128_hbm,
              accn_out,
              si0, si1, si2, si3, di0, di1, di2, di3,
              nr0, nr1, accn_s,
              smi0, smi1, smi2, smi3, smg0, smg1):
    cid = lax.axis_index("c")
    sid = lax.axis_index("s")
    wid = sid * NC + cid
    base = wid * E_PER_W
    row0 = sid * ROWS_PER_TILE

    sidx = [si0, si1, si2, si3]
    didx = [di0, di1, di2, di3]
    nrows = [nr0, nr1]
    semi = [smi0, smi1, smi2, smi3]
    semg = [smg0, smg1]

    pltpu.sync_copy(z128_hbm, accn_s.at[pl.ds(row0, ROWS_PER_TILE), :])
    plsc.subcore_barrier()

    def idx_start(c, s4):
        off = base + c * CHUNK
        pltpu.async_copy(src_hbm.at[pl.ds(off, CHUNK)], sidx[s4], semi[s4])
        pltpu.async_copy(dst_hbm.at[pl.ds(off, CHUNK)], didx[s4], semi[s4])

    def idx_wait(s4):
        pltpu.make_async_copy(src_hbm.at[pl.ds(0, CHUNK)],
                              sidx[s4], semi[s4]).wait()
        pltpu.make_async_copy(dst_hbm.at[pl.ds(0, CHUNK)],
                              didx[s4], semi[s4]).wait()

    def gat_start(s4, b2):
        pltpu.async_copy(nf_hbm.at[sidx[s4]], nrows[b2], semg[b2])

    def gat_wait(b2):
        pltpu.make_async_copy(nf_hbm.at[sidx[0]], nrows[b2], semg[b2]).wait()

    def sca_sync(s4, b2):
        pltpu.sync_copy(nrows[b2], accn_s.at[didx[s4]], add=True)

    def body(c, b, first, last):
        b2, b4 = b % NB, b % NI
        idx_wait(b4)
        if not (last and b >= 2):
            idx_start(c + 2, (b4 + 2) % NI)
        gat_start(b4, b2)
        if not (first and b < 1):
            gat_wait((b2 + 1) % NB)
            sca_sync((b4 + 3) % NI, (b2 + 1) % NB)

    for b in range(NB):
        idx_start(b, b)
    for b in range(NI):
        body(b, b, True, False)

    def outer(g, _):
        for b in range(NI):
            body(g * NI + b, b, False, False)
        return 0
    lax.fori_loop(1, N_FULL // NI - 1, outer, 0)

    for b in range(NI):
        body(N_FULL - NI + b, b, False, True)

    gat_wait((N_FULL - 1) % NB)
    sca_sync((N_FULL - 1) % NI, (N_FULL - 1) % NB)

    plsc.subcore_barrier()
    pltpu.sync_copy(accn_s.at[pl.ds(row0, ROWS_PER_TILE), :],
                    accn_out.at[cid, pl.ds(row0, ROWS_PER_TILE), :])


def _sce_body(ef8_hbm, eidx_hbm, z128_hbm, tmpl_hbm, dep_hbm,
              accec_out,
              didx_v, erows_v, comb_v, accec_s, semi, sems):
    cid = lax.axis_index("c")
    sid = lax.axis_index("s")
    wid = sid * NC + cid
    base = wid * E_PER_W
    row0 = sid * ROWS_PER_TILE

    pltpu.sync_copy(z128_hbm, accec_s.at[pl.ds(row0, ROWS_PER_TILE), :])
    for b4 in range(NB):
        pltpu.sync_copy(tmpl_hbm, comb_v.at[b4])
    plsc.subcore_barrier()

    def in_start(c, s8):
        off = base + c * CHUNK
        row_off = wid * (E_PER_W // 8) + c * (CHUNK // 8)
        pltpu.async_copy(eidx_hbm.at[1, pl.ds(off, CHUNK)],
                         didx_v.at[s8], semi.at[s8])
        pltpu.async_copy(ef8_hbm.at[pl.ds(row_off, CHUNK // 8), :],
                         erows_v.at[s8], semi.at[s8])

    def in_wait(s8):
        pltpu.make_async_copy(eidx_hbm.at[1, pl.ds(0, CHUNK)],
                              didx_v.at[s8], semi.at[s8]).wait()
        pltpu.make_async_copy(ef8_hbm.at[pl.ds(0, CHUNK // 8), :],
                              erows_v.at[s8], semi.at[s8]).wait()

    def sca_start(s8, b4):
        pltpu.async_copy(comb_v.at[b4], accec_s.at[didx_v.at[s8]],
                         sems.at[b4], add=True)

    def sca_wait(s8, b4):
        pltpu.make_async_copy(comb_v.at[b4], accec_s.at[didx_v.at[s8]],
                              sems.at[b4]).wait()

    def body(c, b, first, last):
        b2, b4 = b % NB, b % NI
        in_wait(b4)
        if not (first and b < 2):
            sca_wait((b4 + 2) % NI, b2)
        if not (last and b >= 2):
            in_start(c + 2, (b4 + 2) % NI)

        def _cp(r, _):
            for k in range(8):
                comb_v[b2, r * 8 + k, :DE] = \
                    erows_v[b4, r, k * DE:(k + 1) * DE]
            return 0
        lax.fori_loop(0, CHUNK // 8, _cp, 0, unroll=2)
        sca_start(b4, b2)

    for b in range(NB):
        in_start(b, b)
    for b in range(NI):
        body(b, b, True, False)

    def outer(g, _):
        for b in range(NI):
            body(g * NI + b, b, False, False)
        return 0
    lax.fori_loop(1, N_FULL // NI - 1, outer, 0)

    for b in range(NI):
        body(N_FULL - NI + b, b, False, True)

    for b2 in range(NB):
        sca_wait((N_FULL - NB + b2) % NI, (N_FULL - NB + b2) % NB)

    plsc.subcore_barrier()
    pltpu.sync_copy(accec_s.at[pl.ds(row0, ROWS_PER_TILE), :],
                    accec_out.at[cid, pl.ds(row0, ROWS_PER_TILE), :])


def _sc_segment_sums(nf, ef, srcp, dstp, eidx, z128, tmpl):
    mesh = plsc.VectorSubcoreMesh(core_axis_name="c", subcore_axis_name="s")
    fn = pl.kernel(
        _scn_body,
        out_type=jax.ShapeDtypeStruct((NC, NPAD, DIN), jnp.float32),
        mesh=mesh,
        scratch_types=(
            [pltpu.VMEM((CHUNK,), jnp.int32)] * 8
            + [pltpu.VMEM((CHUNK, DIN), jnp.float32)] * 2
            + [pltpu.VMEM_SHARED((NPAD, DIN), jnp.float32)]
            + [pltpu.SemaphoreType.DMA] * 6
        ),
    )
    accn = fn(nf, srcp, dstp, z128)
    fe = pl.kernel(
        _sce_body,
        out_type=jax.ShapeDtypeStruct((NC, NPAD, DIN), jnp.float32),
        mesh=mesh,
        scratch_types=[
            pltpu.VMEM((NI, CHUNK), jnp.int32),
            pltpu.VMEM((NI, CHUNK // 8, DIN), jnp.float32),
            pltpu.VMEM((NB, CHUNK, DIN), jnp.float32),
            pltpu.VMEM_SHARED((NPAD, DIN), jnp.float32),
            pltpu.SemaphoreType.DMA((NI,)),
            pltpu.SemaphoreType.DMA((NB,)),
        ],
    )
    accec = fe(ef, eidx, z128, tmpl, accn)
    return accn, accec


TC_R = 1000


def _tc_body(nf_ref, accn_ref, accec_ref,
             w1t_ref, w2t_ref, bmsg_ref, wa1t_ref, wa2t_ref, bapp_ref,
             out_ref):
    sn = accn_ref[0] + accn_ref[1]
    ec = accec_ref[0] + accec_ref[1]
    se = ec[:, :DE]
    cnt = ec[:, DE:DE + 1]
    inv = 1.0 / jnp.maximum(cnt, 1.0)
    hn = lax.dot_general(sn, w1t_ref[...], (((1,), (0,)), ((), ())),
                         precision=lax.Precision.HIGHEST)
    hn += lax.dot_general(se, w2t_ref[...], (((1,), (0,)), ((), ())),
                          precision=lax.Precision.HIGHEST)
    hn = (hn + cnt * bmsg_ref[...]) * inv
    out = lax.dot_general(nf_ref[...], wa1t_ref[...], (((1,), (0,)), ((), ())),
                          precision=lax.Precision.HIGHEST)
    out += lax.dot_general(hn, wa2t_ref[...], (((1,), (0,)), ((), ())),
                           precision=lax.Precision.HIGHEST)
    out_ref[...] = jnp.maximum(out + bapp_ref[...], 0.0)


def _tc_apply(nf, accn, accec, w1t, w2t, bmsg, wa1t, wa2t, bapp):
    zero_map = lambda i: (0, 0)
    return pl.pallas_call(
        _tc_body,
        grid=(N // TC_R,),
        in_specs=[
            pl.BlockSpec((TC_R, DIN), lambda i: (i, 0)),
            pl.BlockSpec((NC, TC_R, DIN), lambda i: (0, i, 0)),
            pl.BlockSpec((NC, TC_R, DIN), lambda i: (0, i, 0)),
            pl.BlockSpec((DIN, DOUT), zero_map),
            pl.BlockSpec((DE, DOUT), zero_map),
            pl.BlockSpec((1, DOUT), zero_map),
            pl.BlockSpec((DIN, DOUT), zero_map),
            pl.BlockSpec((DOUT, DOUT), zero_map),
            pl.BlockSpec((1, DOUT), zero_map),
        ],
        out_specs=pl.BlockSpec((TC_R, DOUT), lambda i: (i, 0)),
        out_shape=jax.ShapeDtypeStruct((N, DOUT), jnp.float32),
    )(nf, accn, accec, w1t, w2t, bmsg, wa1t, wa2t, bapp)


@jax.jit
def kernel(nfeats, efeats, edge_index, W_msg_w, W_msg_b, W_apply_w, W_apply_b):
    nf = nfeats[:, 0, :]
    ef = efeats[:, 0, :]
    npad_e = EPAD - E
    pad_src = jnp.arange(npad_e, dtype=jnp.int32) % N
    srcp = jnp.concatenate([edge_index[0], pad_src])
    pad_dst = N + (jnp.arange(npad_e, dtype=jnp.int32) % (NPAD - N))
    dstp = jnp.concatenate([edge_index[1], pad_dst])
    eidx = jnp.stack([srcp, dstp])
    efp = jnp.concatenate([ef, jnp.zeros((npad_e, DE), jnp.float32)])
    ef8 = efp.reshape(EPAD // 8, 8 * DE)
    z128 = jnp.zeros((ROWS_PER_TILE, DIN), jnp.float32)
    tmpl = jnp.zeros((CHUNK, DIN), jnp.float32).at[:, DE].set(1.0)
    accn, accec = _sc_segment_sums(nf, ef8, srcp, dstp, eidx, z128, tmpl)
    w1t = W_msg_w[:, :DIN].T
    w2t = W_msg_w[:, DIN:].T
    wa1t = W_apply_w[:, :DIN].T
    wa2t = W_apply_w[:, DIN:].T
    out = _tc_apply(nf, accn, accec, w1t, w2t,
                    W_msg_b[None, :], wa1t, wa2t, W_apply_b[None, :])
    return out[:, None, :]
